# Initial kernel scaffold; baseline (speedup 1.0000x reference)
import jax
import jax.numpy as jnp
from jax.experimental import pallas as pl


def kernel(x, edge_index, W1, b1, W2, b2, templates, templates_features, W_lin, b_lin):
    raise NotImplementedError("write your pallas kernel here")


# speedup vs baseline: 20.3583x; 20.3583x over previous
import functools

import jax
import jax.numpy as jnp
from jax.experimental import pallas as pl
from jax.experimental.pallas import tpu as pltpu
from jax.experimental.pallas import tpu_sc as plsc

_NC = 2
_NS = 16
_NW = _NC * _NS
_K = 80


def _pad_rows(n):
    return 128 * ((n + 127) // 128)


def _seg_sum_sc(g, src3, dst3, zeros, width):
    npad = zeros.shape[0]
    nch = src3.shape[1]
    rpt = npad // _NS
    mesh = plsc.VectorSubcoreMesh(core_axis_name="c", subcore_axis_name="s")

    @functools.partial(
        pl.kernel,
        out_type=jax.ShapeDtypeStruct((_NC, npad, width), jnp.float32),
        mesh=mesh,
        compiler_params=pltpu.CompilerParams(use_tc_tiling_on_sc=False),
        scratch_types=[
            pltpu.VMEM((nch, _K), jnp.int32),
            pltpu.VMEM((nch, _K), jnp.int32),
            pltpu.VMEM((_K, width), jnp.float32),
            pltpu.VMEM_SHARED((npad, width), jnp.float32),
        ],
    )
    def run(g_hbm, src_hbm, dst_hbm, z_hbm, out_hbm, src_v, dst_v, rows_v, acc_sh):
        c = jax.lax.axis_index("c")
        s = jax.lax.axis_index("s")
        wid = c * _NS + s
        pltpu.sync_copy(src_hbm.at[wid], src_v)
        pltpu.sync_copy(dst_hbm.at[wid], dst_v)
        pltpu.sync_copy(z_hbm.at[pl.ds(s * rpt, rpt)], acc_sh.at[pl.ds(s * rpt, rpt)])
        plsc.subcore_barrier()

        @pl.loop(0, nch)
        def _(j):
            pltpu.sync_copy(g_hbm.at[src_v.at[j]], rows_v)
            pltpu.sync_copy(rows_v, acc_sh.at[dst_v.at[j]], add=True)

        plsc.subcore_barrier()
        pltpu.sync_copy(acc_sh.at[pl.ds(s * rpt, rpt)],
                        out_hbm.at[c, pl.ds(s * rpt, rpt)])

    return run(g, src3, dst3, zeros)


def _deg_sc(dst3, ones, zeros):
    npad = zeros.shape[0]
    nch = dst3.shape[1]
    rpt = npad // _NS
    mesh = plsc.VectorSubcoreMesh(core_axis_name="c", subcore_axis_name="s")

    @functools.partial(
        pl.kernel,
        out_type=jax.ShapeDtypeStruct((_NC, npad, 8), jnp.float32),
        mesh=mesh,
        compiler_params=pltpu.CompilerParams(use_tc_tiling_on_sc=False),
        scratch_types=[
            pltpu.VMEM((nch, _K), jnp.int32),
            pltpu.VMEM((_K, 8), jnp.float32),
            pltpu.VMEM_SHARED((npad, 8), jnp.float32),
        ],
    )
    def run(dst_hbm, ones_hbm, z_hbm, out_hbm, dst_v, ones_v, acc_sh):
        c = jax.lax.axis_index("c")
        s = jax.lax.axis_index("s")
        wid = c * _NS + s
        pltpu.sync_copy(dst_hbm.at[wid], dst_v)
        pltpu.sync_copy(ones_hbm, ones_v)
        pltpu.sync_copy(z_hbm.at[pl.ds(s * rpt, rpt)], acc_sh.at[pl.ds(s * rpt, rpt)])
        plsc.subcore_barrier()

        @pl.loop(0, nch)
        def _(j):
            pltpu.sync_copy(ones_v, acc_sh.at[dst_v.at[j]], add=True)

        plsc.subcore_barrier()
        pltpu.sync_copy(acc_sh.at[pl.ds(s * rpt, rpt)],
                        out_hbm.at[c, pl.ds(s * rpt, rpt)])

    return run(dst3, ones, zeros)


def _tc1_body(x_ref, w1_ref, degp_ref, g0_ref, dinv_ref, deg_ref):
    n = x_ref.shape[0]
    deg = degp_ref[0, 0:n, 0:1] + degp_ref[1, 0:n, 0:1] + 1.0
    dinv = jax.lax.rsqrt(deg)
    h0 = jnp.dot(x_ref[...], w1_ref[...], preferred_element_type=jnp.float32)
    g0_ref[...] = h0 * dinv
    dinv_ref[...] = dinv
    deg_ref[...] = deg


def _tc2_body(s1p_ref, g0_ref, dinv_ref, b1_ref, w2_ref, g1_ref):
    dinv = dinv_ref[...]
    n = g0_ref.shape[0]
    s1 = s1p_ref[0, 0:n] + s1p_ref[1, 0:n] + g0_ref[...]
    h1 = jnp.maximum(dinv * s1 + b1_ref[...], 0.0)
    g1_ref[...] = jnp.dot(h1, w2_ref[...],
                          preferred_element_type=jnp.float32) * dinv


def _tc3_body(s2p_ref, g1_ref, dinv_ref, b2_ref, g2_ref):
    n = g1_ref.shape[0]
    h2 = dinv_ref[...] * (s2p_ref[0, 0:n] + s2p_ref[1, 0:n] + g1_ref[...]) + b2_ref[...]
    sq = jnp.sum(h2 * h2, axis=1, keepdims=True)
    g2_ref[:, 0:64] = h2
    g2_ref[:, 64:65] = sq
    g2_ref[:, 65:80] = jnp.zeros((n, 15), jnp.float32)


def _tc4_body(s3p_ref, g2_ref, deg_ref, t2_ref, tf_ref, wlin_ref, blin_ref,
              out_ref):
    nt = tf_ref.shape[0]
    ntn = tf_ref.shape[1]
    n = g2_ref.shape[0]
    s3 = s3p_ref[0, 0:n] + s3p_ref[1, 0:n] + g2_ref[...]
    mean = s3 / deg_ref[...]
    mean_x = mean[:, 0:64]
    mean_sq = mean[:, 64:65]

    mf = tf_ref[:, 0, :]
    mfsq = jnp.sum(tf_ref[:, 0, :] ** 2, axis=1)
    for k in range(1, ntn):
        tk = tf_ref[:, k, :]
        mf = mf + tk
        mfsq = mfsq + jnp.sum(tk * tk, axis=1)
    mf = mf * (1.0 / ntn)
    mfsq = (mfsq * (1.0 / ntn)).reshape(1, nt)
    struct = (jnp.sum(t2_ref[...] ** 2, axis=1) / t2_ref.shape[1]).reshape(1, nt)

    cross = jax.lax.dot_general(mean_x, mf, (((1,), (1,)), ((), ())),
                                preferred_element_type=jnp.float32)
    feat = mean_sq + mfsq - 2.0 * cross
    y = 0.5 * feat + 0.5 * struct

    h2 = g2_ref[:, 0:64]
    out = (jnp.dot(h2, wlin_ref[0:64], preferred_element_type=jnp.float32)
           + jnp.dot(y, wlin_ref[64:80], preferred_element_type=jnp.float32)
           + blin_ref[...])
    out_ref[...] = jnp.maximum(out, 0.0)


def _tc_call(body, out_shapes, *args):
    return pl.pallas_call(
        body,
        out_shape=[jax.ShapeDtypeStruct(s, jnp.float32) for s in out_shapes],
    )(*args)


def kernel(x, edge_index, W1, b1, W2, b2, templates, templates_features,
           W_lin, b_lin):
    n = x.shape[0]
    e = edge_index.shape[1]
    nt = templates.shape[0]
    nch = e // (_NW * _K)

    src3 = edge_index[0].reshape(_NW, nch, _K)
    dst3 = edge_index[1].reshape(_NW, nch, _K)
    npad = _pad_rows(n)
    ones8 = jnp.ones((_K, 8), jnp.float32)
    z8 = jnp.zeros((npad, 8), jnp.float32)
    z64 = jnp.zeros((npad, 64), jnp.float32)
    z80 = jnp.zeros((npad, 80), jnp.float32)

    degp = _deg_sc(dst3, ones8, z8)
    g0, dinv, deg = _tc_call(_tc1_body, [(n, 64), (n, 1), (n, 1)],
                             x, W1, degp)
    s1p = _seg_sum_sc(g0, src3, dst3, z64, 64)
    (g1,) = _tc_call(_tc2_body, [(n, 64)], s1p, g0, dinv, b1, W2)
    s2p = _seg_sum_sc(g1, src3, dst3, z64, 64)
    (g2,) = _tc_call(_tc3_body, [(n, 80)], s2p, g1, dinv, b2)
    s3p = _seg_sum_sc(g2, src3, dst3, z80, 80)
    (out,) = _tc_call(_tc4_body, [(n, W_lin.shape[1])],
                      s3p, g2, deg, templates.reshape(nt, -1),
                      templates_features, W_lin, b_lin)
    return out


# speedup vs baseline: 35.8116x; 1.7591x over previous
import functools

import jax
import jax.numpy as jnp
from jax.experimental import pallas as pl
from jax.experimental.pallas import tpu as pltpu
from jax.experimental.pallas import tpu_sc as plsc

_NC = 2
_NS = 16
_NW = _NC * _NS
_K = 125
_NBUF = 4


def _pad_rows(n):
    return 128 * ((n + 127) // 128)


def _seg_sum_sc(g, src3, dst3, zeros, width):
    npad = zeros.shape[0]
    nch = src3.shape[1]
    rpt = npad // _NS
    mesh = plsc.VectorSubcoreMesh(core_axis_name="c", subcore_axis_name="s")

    @functools.partial(
        pl.kernel,
        out_type=jax.ShapeDtypeStruct((_NC, npad, width), jnp.float32),
        mesh=mesh,
        compiler_params=pltpu.CompilerParams(use_tc_tiling_on_sc=False),
        scratch_types=[
            pltpu.VMEM((nch, _K), jnp.int32),
            pltpu.VMEM((nch, _K), jnp.int32),
            pltpu.VMEM((_NBUF, _K, width), jnp.float32),
        ] + [pltpu.SemaphoreType.DMA] * (2 * _NBUF) + [
            pltpu.VMEM_SHARED((npad, width), jnp.float32),
        ],
    )
    def run(g_hbm, src_hbm, dst_hbm, z_hbm, out_hbm, src_v, dst_v, rows_v,
            *rest):
        gsem = rest[:_NBUF]
        ssem = rest[_NBUF:2 * _NBUF]
        acc_sh = rest[2 * _NBUF]
        c = jax.lax.axis_index("c")
        s = jax.lax.axis_index("s")
        wid = c * _NS + s
        pltpu.sync_copy(src_hbm.at[wid], src_v)
        pltpu.sync_copy(dst_hbm.at[wid], dst_v)
        pltpu.sync_copy(z_hbm.at[pl.ds(s * rpt, rpt)], acc_sh.at[pl.ds(s * rpt, rpt)])
        plsc.subcore_barrier()

        for b in range(_NBUF):
            pltpu.async_copy(g_hbm.at[src_v.at[b]], rows_v.at[b], gsem[b])

        @pl.loop(0, nch, step=_NBUF)
        def _(j0):
            for b in range(_NBUF):
                pltpu.make_async_copy(g_hbm.at[src_v.at[b]], rows_v.at[b],
                                      gsem[b]).wait()
                pltpu.async_copy(rows_v.at[b], acc_sh.at[dst_v.at[j0 + b]],
                                 ssem[b], add=True)
            for b in range(_NBUF):
                pltpu.make_async_copy(rows_v.at[b],
                                      acc_sh.at[dst_v.at[j0 + b]],
                                      ssem[b]).wait()

                @pl.when(j0 + b + _NBUF < nch)
                def _():
                    pltpu.async_copy(g_hbm.at[src_v.at[j0 + b + _NBUF]],
                                     rows_v.at[b], gsem[b])

        plsc.subcore_barrier()
        pltpu.sync_copy(acc_sh.at[pl.ds(s * rpt, rpt)],
                        out_hbm.at[c, pl.ds(s * rpt, rpt)])

    return run(g, src3, dst3, zeros)


def _deg_sc(dst3, ones, zeros):
    npad = zeros.shape[0]
    nch = dst3.shape[1]
    rpt = npad // _NS
    mesh = plsc.VectorSubcoreMesh(core_axis_name="c", subcore_axis_name="s")

    @functools.partial(
        pl.kernel,
        out_type=jax.ShapeDtypeStruct((_NC, npad, 8), jnp.float32),
        mesh=mesh,
        compiler_params=pltpu.CompilerParams(use_tc_tiling_on_sc=False),
        scratch_types=[
            pltpu.VMEM((nch, _K), jnp.int32),
            pltpu.VMEM((_K, 8), jnp.float32),
            pltpu.SemaphoreType.DMA,
            pltpu.VMEM_SHARED((npad, 8), jnp.float32),
        ],
    )
    def run(dst_hbm, ones_hbm, z_hbm, out_hbm, dst_v, ones_v, sem, acc_sh):
        c = jax.lax.axis_index("c")
        s = jax.lax.axis_index("s")
        wid = c * _NS + s
        pltpu.sync_copy(dst_hbm.at[wid], dst_v)
        pltpu.sync_copy(ones_hbm, ones_v)
        pltpu.sync_copy(z_hbm.at[pl.ds(s * rpt, rpt)], acc_sh.at[pl.ds(s * rpt, rpt)])
        plsc.subcore_barrier()

        @pl.loop(0, nch, step=8)
        def _(j0):
            for i in range(8):
                pltpu.async_copy(ones_v, acc_sh.at[dst_v.at[j0 + i]], sem,
                                 add=True)
            for i in range(8):
                pltpu.make_async_copy(ones_v, acc_sh.at[dst_v.at[j0 + i]],
                                      sem).wait()

        plsc.subcore_barrier()
        pltpu.sync_copy(acc_sh.at[pl.ds(s * rpt, rpt)],
                        out_hbm.at[c, pl.ds(s * rpt, rpt)])

    return run(dst3, ones, zeros)


def _tc1_body(x_ref, w1_ref, degp_ref, g0_ref, dinv_ref, deg_ref):
    n = x_ref.shape[0]
    deg = degp_ref[0, 0:n, 0:1] + degp_ref[1, 0:n, 0:1] + 1.0
    dinv = jax.lax.rsqrt(deg)
    h0 = jnp.dot(x_ref[...], w1_ref[...], preferred_element_type=jnp.float32)
    g0_ref[...] = h0 * dinv
    dinv_ref[...] = dinv
    deg_ref[...] = deg


def _tc2_body(s1p_ref, g0_ref, dinv_ref, b1_ref, w2_ref, g1_ref):
    dinv = dinv_ref[...]
    n = g0_ref.shape[0]
    s1 = s1p_ref[0, 0:n] + s1p_ref[1, 0:n] + g0_ref[...]
    h1 = jnp.maximum(dinv * s1 + b1_ref[...], 0.0)
    g1_ref[...] = jnp.dot(h1, w2_ref[...],
                          preferred_element_type=jnp.float32) * dinv


def _tc3_body(s2p_ref, g1_ref, dinv_ref, b2_ref, g2_ref):
    n = g1_ref.shape[0]
    h2 = dinv_ref[...] * (s2p_ref[0, 0:n] + s2p_ref[1, 0:n] + g1_ref[...]) + b2_ref[...]
    sq = jnp.sum(h2 * h2, axis=1, keepdims=True)
    g2_ref[:, 0:64] = h2
    g2_ref[:, 64:65] = sq
    g2_ref[:, 65:80] = jnp.zeros((n, 15), jnp.float32)


def _tc4_body(s3p_ref, g2_ref, deg_ref, t2_ref, tf_ref, wlin_ref, blin_ref,
              out_ref):
    nt = tf_ref.shape[0]
    ntn = tf_ref.shape[1]
    n = g2_ref.shape[0]
    s3 = s3p_ref[0, 0:n] + s3p_ref[1, 0:n] + g2_ref[...]
    mean = s3 / deg_ref[...]
    mean_x = mean[:, 0:64]
    mean_sq = mean[:, 64:65]

    mf = tf_ref[:, 0, :]
    mfsq = jnp.sum(tf_ref[:, 0, :] ** 2, axis=1)
    for k in range(1, ntn):
        tk = tf_ref[:, k, :]
        mf = mf + tk
        mfsq = mfsq + jnp.sum(tk * tk, axis=1)
    mf = mf * (1.0 / ntn)
    mfsq = (mfsq * (1.0 / ntn)).reshape(1, nt)
    struct = (jnp.sum(t2_ref[...] ** 2, axis=1) / t2_ref.shape[1]).reshape(1, nt)

    cross = jax.lax.dot_general(mean_x, mf, (((1,), (1,)), ((), ())),
                                preferred_element_type=jnp.float32)
    feat = mean_sq + mfsq - 2.0 * cross
    y = 0.5 * feat + 0.5 * struct

    h2 = g2_ref[:, 0:64]
    out = (jnp.dot(h2, wlin_ref[0:64], preferred_element_type=jnp.float32)
           + jnp.dot(y, wlin_ref[64:80], preferred_element_type=jnp.float32)
           + blin_ref[...])
    out_ref[...] = jnp.maximum(out, 0.0)


def _tc_call(body, out_shapes, *args):
    return pl.pallas_call(
        body,
        out_shape=[jax.ShapeDtypeStruct(s, jnp.float32) for s in out_shapes],
    )(*args)


def kernel(x, edge_index, W1, b1, W2, b2, templates, templates_features,
           W_lin, b_lin):
    n = x.shape[0]
    e = edge_index.shape[1]
    nt = templates.shape[0]
    nch = e // (_NW * _K)

    src3 = edge_index[0].reshape(_NW, nch, _K)
    dst3 = edge_index[1].reshape(_NW, nch, _K)
    npad = _pad_rows(n)
    ones8 = jnp.ones((_K, 8), jnp.float32)
    z8 = jnp.zeros((npad, 8), jnp.float32)
    z64 = jnp.zeros((npad, 64), jnp.float32)
    z80 = jnp.zeros((npad, 80), jnp.float32)

    degp = _deg_sc(dst3, ones8, z8)
    g0, dinv, deg = _tc_call(_tc1_body, [(n, 64), (n, 1), (n, 1)],
                             x, W1, degp)
    s1p = _seg_sum_sc(g0, src3, dst3, z64, 64)
    (g1,) = _tc_call(_tc2_body, [(n, 64)], s1p, g0, dinv, b1, W2)
    s2p = _seg_sum_sc(g1, src3, dst3, z64, 64)
    (g2,) = _tc_call(_tc3_body, [(n, 80)], s2p, g1, dinv, b2)
    s3p = _seg_sum_sc(g2, src3, dst3, z80, 80)
    (out,) = _tc_call(_tc4_body, [(n, W_lin.shape[1])],
                      s3p, g2, deg, templates.reshape(nt, -1),
                      templates_features, W_lin, b_lin)
    return out
